# baseline (speedup 1.0000x reference)
import jax
import jax.numpy as jnp
from jax.experimental import pallas as pl
from jax.experimental.pallas import tpu as pltpu

_BETA = 1.0
_GAMMA = 0.5
_NSEG = 512
_NEG = -1e30
_K = 32


def _body(h_ref, b_ref, mk_ref, pp_ref, ss_ref, w_ref, sid_ref, bs_ref,
          out_ref, m_ref, s_ref, acc_ref):
    i = pl.program_id(0)
    nb = pl.num_programs(0)
    r = h_ref.shape[0]

    h = h_ref[...]
    hb = h.astype(jnp.bfloat16)
    lrow = jax.lax.dot_general(w_ref[...], h, (((1,), (1,)), ((), ())),
                               preferred_element_type=jnp.float32)
    batch = b_ref[0]
    mask = mk_ref[0] > 0.5
    ml = jnp.where(mask, lrow + _BETA * pp_ref[0] + _GAMMA * ss_ref[0], _NEG)

    @pl.when(i == 0)
    def _init():
        m_ref[...] = jnp.full(m_ref.shape, _NEG, jnp.float32)
        s_ref[...] = jnp.zeros(s_ref.shape, jnp.float32)
        acc_ref[...] = jnp.zeros(acc_ref.shape, jnp.float32)

    first = bs_ref[0, 0, 0]
    last = bs_ref[0, 0, r - 1]
    lo = (first // 8) * 8
    nw = (last - lo) // _K + 1

    def _window(w, _):
        lo_w = pl.multiple_of(lo + w * _K, 8)
        sid = sid_ref[pl.ds(lo_w, _K), :]
        oh = sid == batch
        m_old = m_ref[pl.ds(lo_w, _K), :]
        bmax = jnp.max(jnp.where(oh, ml, _NEG), axis=1, keepdims=True)
        m_new = jnp.maximum(m_old, bmax)
        alpha = jnp.exp(m_old - m_new)
        rowm = jnp.max(jnp.where(oh, m_new, _NEG), axis=0, keepdims=True)
        e = jnp.where(mask, jnp.exp(ml - rowm), 0.0)
        ohw = jnp.where(oh, e, 0.0)
        bsum = jnp.sum(ohw, axis=1, keepdims=True)
        s_ref[pl.ds(lo_w, _K), :] = s_ref[pl.ds(lo_w, _K), :] * alpha + bsum
        m_ref[pl.ds(lo_w, _K), :] = m_new

        mm = jnp.dot(ohw.astype(jnp.bfloat16), hb,
                     preferred_element_type=jnp.float32)
        acc_ref[pl.ds(lo_w, _K), :] = acc_ref[pl.ds(lo_w, _K), :] * alpha + mm
        return ()

    jax.lax.fori_loop(0, nw, _window, ())

    @pl.when(i == nb - 1)
    def _finish():
        d = s_ref[: _NSEG, :]
        out_ref[...] = acc_ref[: _NSEG, :] / jnp.where(d > 0, d, 1.0)


def _pick_block(n):
    for r in range(16384, 7, -8):
        if n % r == 0:
            return r
    return n


@jax.jit
def kernel(h, batch, node_mask, paratope_prob, sasa_prior, W):
    n, d = h.shape
    r = _pick_block(n)
    nb = n // r

    b3 = batch.astype(jnp.int32).reshape(nb, 1, r)
    mk3 = node_mask.astype(jnp.float32).reshape(nb, 1, r)
    pp3 = paratope_prob.astype(jnp.float32).reshape(nb, 1, r)
    ss3 = sasa_prior.astype(jnp.float32).reshape(nb, 1, r)
    w1 = W.astype(jnp.float32).reshape(1, d)
    sids = jnp.arange(_NSEG + _K, dtype=jnp.int32).reshape(_NSEG + _K, 1)

    row_spec = pl.BlockSpec((1, 1, r), lambda i: (i, 0, 0))
    out = pl.pallas_call(
        _body,
        grid=(nb,),
        in_specs=[
            pl.BlockSpec((r, d), lambda i: (i, 0)),
            row_spec, row_spec, row_spec, row_spec,
            pl.BlockSpec((1, d), lambda i: (0, 0)),
            pl.BlockSpec((_NSEG + _K, 1), lambda i: (0, 0)),
            pl.BlockSpec((1, 1, r), lambda i: (i, 0, 0),
                         memory_space=pltpu.SMEM),
        ],
        out_specs=pl.BlockSpec((_NSEG, d), lambda i: (0, 0)),
        out_shape=jax.ShapeDtypeStruct((_NSEG, d), jnp.float32),
        scratch_shapes=[
            pltpu.VMEM((_NSEG + _K, 1), jnp.float32),
            pltpu.VMEM((_NSEG + _K, 1), jnp.float32),
            pltpu.VMEM((_NSEG + _K, d), jnp.float32),
        ],
    )(h.astype(jnp.float32), b3, mk3, pp3, ss3, w1, sids, b3)
    return out


# speedup vs baseline: 1.1227x; 1.1227x over previous
import jax
import jax.numpy as jnp
from jax.experimental import pallas as pl
from jax.experimental.pallas import tpu as pltpu

_BETA = 1.0
_GAMMA = 0.5
_NSEG = 512
_NEG = -1e30
_K = 32


def _body(h_ref, b_ref, mk_ref, pp_ref, ss_ref, w_ref, sid_ref, bs_ref,
          out_ref, m_ref, s_ref, acc_ref):
    i = pl.program_id(0)
    nb = pl.num_programs(0)
    r = h_ref.shape[0]

    h = h_ref[...]
    hb = h.astype(jnp.bfloat16)
    lrow = jax.lax.dot_general(w_ref[...], h, (((1,), (1,)), ((), ())),
                               preferred_element_type=jnp.float32)
    batch = b_ref[0]
    mask = mk_ref[0] > 0.5
    ml = jnp.where(mask, lrow + _BETA * pp_ref[0] + _GAMMA * ss_ref[0], _NEG)

    @pl.when(i == 0)
    def _init():
        m_ref[...] = jnp.full(m_ref.shape, _NEG, jnp.float32)
        s_ref[...] = jnp.zeros(s_ref.shape, jnp.float32)
        acc_ref[...] = jnp.zeros(acc_ref.shape, jnp.float32)

    first = bs_ref[0, 0, 0]
    last = bs_ref[0, 0, r - 1]
    lo = (first // 8) * 8
    nw = (last - lo) // _K + 1

    def _window(w, _):
        lo_w = pl.multiple_of(lo + w * _K, 8)
        sid = sid_ref[pl.ds(lo_w, _K), :]
        oh = sid == batch
        m_old = m_ref[pl.ds(lo_w, _K), :]
        bmax = jnp.max(jnp.where(oh, ml, _NEG), axis=1, keepdims=True)
        m_new = jnp.maximum(m_old, bmax)
        alpha = jnp.exp(m_old - m_new)
        rowm = jnp.max(jnp.where(oh, m_new, _NEG), axis=0, keepdims=True)
        e = jnp.where(mask, jnp.exp(ml - rowm), 0.0)
        ohw = jnp.where(oh, e, 0.0)
        bsum = jnp.sum(ohw, axis=1, keepdims=True)
        s_ref[pl.ds(lo_w, _K), :] = s_ref[pl.ds(lo_w, _K), :] * alpha + bsum
        m_ref[pl.ds(lo_w, _K), :] = m_new

        mm = jnp.dot(ohw.astype(jnp.bfloat16), hb,
                     preferred_element_type=jnp.float32)
        acc_ref[pl.ds(lo_w, _K), :] = acc_ref[pl.ds(lo_w, _K), :] * alpha + mm
        return ()

    jax.lax.fori_loop(0, nw, _window, ())

    @pl.when(i == nb - 1)
    def _finish():
        d = s_ref[: _NSEG, :]
        out_ref[...] = acc_ref[: _NSEG, :] / jnp.where(d > 0, d, 1.0)


def _pick_block(n):
    for r in range(8192, 7, -8):
        if n % r == 0:
            return r
    return n


@jax.jit
def kernel(h, batch, node_mask, paratope_prob, sasa_prior, W):
    n, d = h.shape
    r = _pick_block(n)
    nb = n // r

    b3 = batch.astype(jnp.int32).reshape(nb, 1, r)
    mk3 = node_mask.astype(jnp.float32).reshape(nb, 1, r)
    pp3 = paratope_prob.astype(jnp.float32).reshape(nb, 1, r)
    ss3 = sasa_prior.astype(jnp.float32).reshape(nb, 1, r)
    w1 = W.astype(jnp.float32).reshape(1, d)
    sids = jnp.arange(_NSEG + _K, dtype=jnp.int32).reshape(_NSEG + _K, 1)

    row_spec = pl.BlockSpec((1, 1, r), lambda i: (i, 0, 0))
    out = pl.pallas_call(
        _body,
        grid=(nb,),
        in_specs=[
            pl.BlockSpec((r, d), lambda i: (i, 0)),
            row_spec, row_spec, row_spec, row_spec,
            pl.BlockSpec((1, d), lambda i: (0, 0)),
            pl.BlockSpec((_NSEG + _K, 1), lambda i: (0, 0)),
            pl.BlockSpec((1, 1, r), lambda i: (i, 0, 0),
                         memory_space=pltpu.SMEM),
        ],
        out_specs=pl.BlockSpec((_NSEG, d), lambda i: (0, 0)),
        out_shape=jax.ShapeDtypeStruct((_NSEG, d), jnp.float32),
        scratch_shapes=[
            pltpu.VMEM((_NSEG + _K, 1), jnp.float32),
            pltpu.VMEM((_NSEG + _K, 1), jnp.float32),
            pltpu.VMEM((_NSEG + _K, d), jnp.float32),
        ],
    )(h.astype(jnp.float32), b3, mk3, pp3, ss3, w1, sids, b3)
    return out
